# baseline (speedup 1.0000x reference)
import functools

import jax
import jax.numpy as jnp
import numpy as np
from jax.experimental import pallas as pl

_VOCABS = (6, 4, 4, 4, 6, 4, 4)
_OFF = (0, 6, 10, 14, 18, 24, 28)
_EMBED = 8
_B = 16384
_BS = 2048


def _body(uc_ref, un_ref, r_ref, cmp_ref, tblk_ref, w1_ref, w1n_ref, b1_ref,
          w2_ref, b2_ref, w3_ref, b3_ref, out_ref):
    ucf = uc_ref[...].astype(jnp.float32)
    ucx = jnp.dot(ucf, r_ref[...], preferred_element_type=jnp.float32)
    m = (ucx == cmp_ref[...]).astype(jnp.float32)
    g = jnp.dot(tblk_ref[...], w1_ref[...], preferred_element_type=jnp.float32)
    h = (jnp.dot(m, g, preferred_element_type=jnp.float32)
         + jnp.dot(un_ref[...], w1n_ref[...], preferred_element_type=jnp.float32)
         + b1_ref[...])
    h = jnp.maximum(h, 0.0)
    h = jnp.dot(h, w2_ref[...], preferred_element_type=jnp.float32) + b2_ref[...]
    h = jnp.maximum(h, 0.0)
    out_ref[...] = (jnp.dot(h, w3_ref[...], preferred_element_type=jnp.float32)
                    + b3_ref[...])


@functools.partial(jax.jit, static_argnames=("interpret",))
def kernel(u_cat, u_num, T_light, T_hum, T_care, T_size, T_climate, T_water,
           T_care_freq, W1, b1, W2, b2, W3, b3, interpret=False):
    tables = [T_light, T_hum, T_care, T_size, T_climate, T_water, T_care_freq]
    tblk = jnp.zeros((32, 64), jnp.float32)
    for i, (t, o) in enumerate(zip(tables, _OFF)):
        tblk = tblk.at[o:o + _VOCABS[i], 8 * i:8 * i + _EMBED].set(t)
    w1p = jnp.zeros((64, 128), jnp.float32).at[:58].set(W1)
    w1n = jnp.zeros((8, 128), jnp.float32).at[:2].set(W1[56:58])

    r_np = np.zeros((8, 32), np.float32)
    cmp_np = np.zeros((1, 32), np.float32)
    for i, (o, v) in enumerate(zip(_OFF, _VOCABS)):
        r_np[i, o:o + v] = 1.0
        cmp_np[0, o:o + v] = np.arange(v, dtype=np.float32)
    r = jnp.asarray(r_np)
    cmp = jnp.asarray(cmp_np)

    ucp = jnp.zeros((_B, 8), jnp.int32).at[:, :7].set(u_cat.astype(jnp.int32))
    unp = jnp.zeros((_B, 8), jnp.float32).at[:, :2].set(u_num)

    grid = (_B // _BS,)
    out = pl.pallas_call(
        _body,
        grid=grid,
        in_specs=[
            pl.BlockSpec((_BS, 8), lambda i: (i, 0)),
            pl.BlockSpec((_BS, 8), lambda i: (i, 0)),
            pl.BlockSpec((8, 32), lambda i: (0, 0)),
            pl.BlockSpec((1, 32), lambda i: (0, 0)),
            pl.BlockSpec((32, 64), lambda i: (0, 0)),
            pl.BlockSpec((64, 128), lambda i: (0, 0)),
            pl.BlockSpec((8, 128), lambda i: (0, 0)),
            pl.BlockSpec((1, 128), lambda i: (0, 0)),
            pl.BlockSpec((128, 128), lambda i: (0, 0)),
            pl.BlockSpec((1, 128), lambda i: (0, 0)),
            pl.BlockSpec((128, 64), lambda i: (0, 0)),
            pl.BlockSpec((1, 64), lambda i: (0, 0)),
        ],
        out_specs=pl.BlockSpec((_BS, 64), lambda i: (i, 0)),
        out_shape=jax.ShapeDtypeStruct((_B, 64), jnp.float32),
        interpret=interpret,
    )(ucp, unp, r, cmp, tblk, w1p, w1n, b1.reshape(1, 128), W2,
      b2.reshape(1, 128), W3, b3.reshape(1, 64))
    return out


# speedup vs baseline: 1.9194x; 1.9194x over previous
import functools

import jax
import jax.numpy as jnp
from jax.experimental import pallas as pl

_VOCABS = (6, 4, 4, 4, 6, 4, 4)
_OFF = (0, 6, 10, 14, 18, 24, 28)
_B = 16384
_BS = 4096


def _body(uc_ref, un_ref, t0, t1, t2, t3, t4, t5, t6, w1_ref, b1_ref, w2_ref,
          b2_ref, w3_ref, b3_ref, out_ref):
    f32 = jnp.float32
    l8 = jax.lax.broadcasted_iota(jnp.int32, (8, 32), 1)
    s8 = jax.lax.broadcasted_iota(jnp.int32, (8, 32), 0)
    fv = jnp.zeros((8, 32), jnp.int32)
    offv = jnp.zeros((8, 32), jnp.int32)
    for bnd, jump in zip(_OFF[1:], (6, 4, 4, 4, 6, 4)):
        step = (l8 >= bnd).astype(jnp.int32)
        fv = fv + step
        offv = offv + jump * step
    rm = (fv == s8).astype(f32)
    cmpv = (l8 - offv).astype(f32)[0:1]

    ucf = uc_ref[...].astype(f32)
    ucx = jnp.dot(ucf, rm[:7, :], preferred_element_type=f32)
    m = (ucx == cmpv).astype(f32)

    tabs = (t0, t1, t2, t3, t4, t5, t6)
    g = jnp.concatenate(
        [jnp.dot(t[...], w1_ref[8 * i:8 * i + 8, :],
                 preferred_element_type=f32) for i, t in enumerate(tabs)],
        axis=0)
    h = (jnp.dot(m, g, preferred_element_type=f32)
         + jnp.dot(un_ref[...], w1_ref[56:58, :], preferred_element_type=f32)
         + b1_ref[...])
    h = jnp.maximum(h, 0.0)
    h = jnp.dot(h, w2_ref[...], preferred_element_type=f32) + b2_ref[...]
    h = jnp.maximum(h, 0.0)
    out_ref[...] = (jnp.dot(h, w3_ref[...], preferred_element_type=f32)
                    + b3_ref[...])


@functools.partial(jax.jit, static_argnames=("interpret",))
def kernel(u_cat, u_num, T_light, T_hum, T_care, T_size, T_climate, T_water,
           T_care_freq, W1, b1, W2, b2, W3, b3, interpret=False):
    tables = [T_light, T_hum, T_care, T_size, T_climate, T_water, T_care_freq]
    const = lambda s: pl.BlockSpec(s, lambda i: (0,) * len(s))
    grid = (_B // _BS,)
    out = pl.pallas_call(
        _body,
        grid=grid,
        in_specs=[
            pl.BlockSpec((_BS, 7), lambda i: (i, 0)),
            pl.BlockSpec((_BS, 2), lambda i: (i, 0)),
            *[const((v, 8)) for v in _VOCABS],
            const((58, 128)),
            const((1, 128)),
            const((128, 128)),
            const((1, 128)),
            const((128, 64)),
            const((1, 64)),
        ],
        out_specs=pl.BlockSpec((_BS, 64), lambda i: (i, 0)),
        out_shape=jax.ShapeDtypeStruct((_B, 64), jnp.float32),
        interpret=interpret,
    )(u_cat.astype(jnp.int32), u_num, *tables, W1, b1.reshape(1, 128), W2,
      b2.reshape(1, 128), W3, b3.reshape(1, 64))
    return out
